# baseline (speedup 1.0000x reference)
import functools

import jax
import jax.numpy as jnp
from jax import lax
from jax.experimental import pallas as pl
from jax.experimental.pallas import tpu as pltpu
from jax.experimental.pallas import tpu_sc as plsc

VOCAB = 1000000
EMB = 128
WINDOW = 5
HIDDEN = 256
N_TAGS = 50
BATCH = 16384

SPLITS = (10240, 6144)
NW = 32
BM = 1024


def _sc_gather_body(chunk, n_chunks, cpp_shift, bh, off,
                    table_hbm, idxt_hbm, out_hbm,
                    idx_v, rows0, rows1, sem0, sem1):
    c = lax.axis_index("c")
    s = lax.axis_index("s")
    wid = s * 2 + c
    b_per_w = n_chunks * chunk
    base = wid * b_per_w
    cpp_mask = (1 << cpp_shift) - 1
    for k in range(n_chunks):
        gcid = wid * n_chunks + k
        plane = lax.shift_right_logical(gcid, cpp_shift)
        pos = lax.bitwise_and(gcid, cpp_mask) * chunk
        src = plane * BATCH + off + pos
        pltpu.sync_copy(idxt_hbm.at[pl.ds(src, chunk)],
                        idx_v.at[pl.ds(k * chunk, chunk)])
    rows = (rows0, rows1)
    sems = (sem0, sem1)
    descs = [None, None]
    descs[0] = pltpu.async_copy(
        table_hbm.at[idx_v.at[pl.ds(0, chunk)]], rows[0], sems[0]
    )
    for k in range(n_chunks):
        b = k & 1
        if k + 1 < n_chunks:
            descs[1 - b] = pltpu.async_copy(
                table_hbm.at[idx_v.at[pl.ds((k + 1) * chunk, chunk)]],
                rows[1 - b],
                sems[1 - b],
            )
        descs[b].wait()
        pltpu.sync_copy(rows[b], out_hbm.at[pl.ds(base + k * chunk, chunk)])


def _make_sc_gather(bh, off):
    n_idx = bh * WINDOW
    b_per_w = n_idx // NW
    import math
    chunk = math.gcd(b_per_w, bh)
    while chunk > 512:
        chunk //= 2
    n_chunks = b_per_w // chunk
    cpp = bh // chunk
    cpp_shift = cpp.bit_length() - 1
    assert (1 << cpp_shift) == cpp and b_per_w % chunk == 0
    assert chunk % 8 == 0
    mesh = plsc.VectorSubcoreMesh(core_axis_name="c", subcore_axis_name="s")
    run = pl.kernel(
        functools.partial(_sc_gather_body, chunk, n_chunks, cpp_shift, bh, off),
        mesh=mesh,
        out_type=jax.ShapeDtypeStruct((n_idx, EMB), jnp.float32),
        scratch_types=[
            pltpu.VMEM((b_per_w,), jnp.int32),
            pltpu.VMEM((chunk, EMB), jnp.float32),
            pltpu.VMEM((chunk, EMB), jnp.float32),
            pltpu.SemaphoreType.DMA,
            pltpu.SemaphoreType.DMA,
        ],
    )
    return run


_SC_GATHERS = []
_off = 0
for _bh in SPLITS:
    _SC_GATHERS.append(_make_sc_gather(_bh, _off))
    _off += _bh


def _mlp_body(out_prev_ref, rows_ref, w1_ref, b1_ref, w2_ref, b2_ref, out_ref):
    del out_prev_ref
    acc = b1_ref[...] + jnp.dot(
        rows_ref[0], w1_ref[0], preferred_element_type=jnp.float32
    )
    for w in range(1, WINDOW):
        acc = acc + jnp.dot(
            rows_ref[w], w1_ref[w], preferred_element_type=jnp.float32
        )
    h = jnp.tanh(acc)
    out = jnp.dot(h, w2_ref[...], preferred_element_type=jnp.float32) + b2_ref[...]
    out_ref[...] = out[:, :N_TAGS]


def _mlp(bh, block_off, out_prev, rows3, W13, b1, W2p, b2p):
    return pl.pallas_call(
        _mlp_body,
        grid=(bh // BM,),
        in_specs=[
            pl.BlockSpec(memory_space=pl.ANY),
            pl.BlockSpec((WINDOW, BM, EMB), lambda i: (0, i, 0)),
            pl.BlockSpec((WINDOW, EMB, HIDDEN), lambda i: (0, 0, 0)),
            pl.BlockSpec((1, HIDDEN), lambda i: (0, 0)),
            pl.BlockSpec((HIDDEN, 128), lambda i: (0, 0)),
            pl.BlockSpec((1, 128), lambda i: (0, 0)),
        ],
        out_specs=pl.BlockSpec((BM, N_TAGS), lambda i: (i + block_off, 0)),
        out_shape=jax.ShapeDtypeStruct((BATCH, N_TAGS), jnp.float32),
        input_output_aliases={0: 0},
    )(out_prev, rows3, W13, b1, W2p, b2p)


def kernel(x, table, W1, b1, W2, b2):
    idxt = x.astype(jnp.int32).T.reshape(-1)
    W13 = W1.reshape(WINDOW, EMB, HIDDEN)
    W2p = jnp.pad(W2, ((0, 0), (0, 128 - N_TAGS)))
    b2p = jnp.pad(b2, (0, 128 - N_TAGS))
    b1r = b1.reshape(1, -1)
    b2r = b2p.reshape(1, -1)
    rows_list = [
        g(table, idxt).reshape(WINDOW, bh, EMB)
        for g, bh in zip(_SC_GATHERS, SPLITS)
    ]
    out = jnp.zeros((BATCH, N_TAGS), jnp.float32)
    block_off = 0
    for bh, r in zip(SPLITS, rows_list):
        out = _mlp(bh, block_off, out, r, W13, b1r, W2p, b2r)
        block_off += bh // BM
    return out


# speedup vs baseline: 1.0512x; 1.0512x over previous
import functools

import jax
import jax.numpy as jnp
from jax import lax
from jax.experimental import pallas as pl
from jax.experimental.pallas import tpu as pltpu
from jax.experimental.pallas import tpu_sc as plsc

VOCAB = 1000000
EMB = 128
WINDOW = 5
HIDDEN = 256
N_TAGS = 50
BATCH = 16384

N_IDX = BATCH * WINDOW
NW = 32
B_PER_W = N_IDX // NW
CHUNK = 320
N_CHUNKS = B_PER_W // CHUNK
BM = 1024


def _sc_gather_body(table_hbm, idx_hbm, out_hbm, idx_v, rows0, rows1, sem0, sem1):
    c = lax.axis_index("c")
    s = lax.axis_index("s")
    wid = s * 2 + c
    base = wid * B_PER_W
    pltpu.sync_copy(idx_hbm.at[pl.ds(base, B_PER_W)], idx_v)
    rows = (rows0, rows1)
    sems = (sem0, sem1)
    descs = [None, None]
    descs[0] = pltpu.async_copy(
        table_hbm.at[idx_v.at[pl.ds(0, CHUNK)]], rows[0], sems[0]
    )
    for k in range(N_CHUNKS):
        b = k & 1
        if k + 1 < N_CHUNKS:
            descs[1 - b] = pltpu.async_copy(
                table_hbm.at[idx_v.at[pl.ds((k + 1) * CHUNK, CHUNK)]],
                rows[1 - b],
                sems[1 - b],
            )
        descs[b].wait()
        pltpu.sync_copy(rows[b], out_hbm.at[pl.ds(base + k * CHUNK, CHUNK)])


@jax.jit
def _sc_gather(table, idx):
    mesh = plsc.VectorSubcoreMesh(core_axis_name="c", subcore_axis_name="s")
    run = pl.kernel(
        _sc_gather_body,
        mesh=mesh,
        out_type=jax.ShapeDtypeStruct((N_IDX, EMB), jnp.float32),
        scratch_types=[
            pltpu.VMEM((B_PER_W,), jnp.int32),
            pltpu.VMEM((CHUNK, EMB), jnp.float32),
            pltpu.VMEM((CHUNK, EMB), jnp.float32),
            pltpu.SemaphoreType.DMA,
            pltpu.SemaphoreType.DMA,
        ],
    )
    return run(table, idx)


def _mlp_body(rows_ref, w1_ref, b1_ref, w2_ref, b2_ref, out_ref):
    acc = b1_ref[...] + jnp.dot(
        rows_ref[0], w1_ref[0], preferred_element_type=jnp.float32
    )
    for w in range(1, WINDOW):
        acc = acc + jnp.dot(
            rows_ref[w], w1_ref[w], preferred_element_type=jnp.float32
        )
    h = jnp.tanh(acc)
    out = jnp.dot(h, w2_ref[...], preferred_element_type=jnp.float32) + b2_ref[...]
    out_ref[...] = out[:, :N_TAGS]


@jax.jit
def _mlp(rows3, W13, b1, W2p, b2p):
    return pl.pallas_call(
        _mlp_body,
        grid=(BATCH // BM,),
        in_specs=[
            pl.BlockSpec((WINDOW, BM, EMB), lambda i: (0, i, 0)),
            pl.BlockSpec((WINDOW, EMB, HIDDEN), lambda i: (0, 0, 0)),
            pl.BlockSpec((1, HIDDEN), lambda i: (0, 0)),
            pl.BlockSpec((HIDDEN, 128), lambda i: (0, 0)),
            pl.BlockSpec((1, 128), lambda i: (0, 0)),
        ],
        out_specs=pl.BlockSpec((BM, N_TAGS), lambda i: (i, 0)),
        out_shape=jax.ShapeDtypeStruct((BATCH, N_TAGS), jnp.float32),
    )(rows3, W13, b1, W2p, b2p)


def kernel(x, table, W1, b1, W2, b2):
    idx = x.astype(jnp.int32).T.reshape(-1)
    rows = _sc_gather(table, idx)
    rows3 = rows.reshape(WINDOW, BATCH, EMB)
    W13 = W1.reshape(WINDOW, EMB, HIDDEN)
    W2p = jnp.pad(W2, ((0, 0), (0, 128 - N_TAGS)))
    b2p = jnp.pad(b2, (0, 128 - N_TAGS))
    return _mlp(rows3, W13, b1.reshape(1, -1), W2p, b2p.reshape(1, -1))


# speedup vs baseline: 1.0517x; 1.0005x over previous
import functools

import jax
import jax.numpy as jnp
from jax import lax
from jax.experimental import pallas as pl
from jax.experimental.pallas import tpu as pltpu
from jax.experimental.pallas import tpu_sc as plsc

VOCAB = 1000000
EMB = 128
WINDOW = 5
HIDDEN = 256
N_TAGS = 50
BATCH = 16384

N_IDX = BATCH * WINDOW
NW = 32
B_PER_W = N_IDX // NW
CHUNK = 160
N_CHUNKS = B_PER_W // CHUNK
NBUF = 4
BM = 1024


def _sc_gather_body(table_hbm, idx_hbm, out_hbm, idx_v, *bufs):
    rows = bufs[:NBUF]
    sems = bufs[NBUF:]
    c = lax.axis_index("c")
    s = lax.axis_index("s")
    wid = s * 2 + c
    base = wid * B_PER_W
    pltpu.sync_copy(idx_hbm.at[pl.ds(base, B_PER_W)], idx_v)
    descs = [None] * NBUF
    for k in range(NBUF - 1):
        descs[k] = pltpu.async_copy(
            table_hbm.at[idx_v.at[pl.ds(k * CHUNK, CHUNK)]], rows[k], sems[k]
        )
    for k in range(N_CHUNKS):
        b = k % NBUF
        kn = k + NBUF - 1
        if kn < N_CHUNKS:
            bn = kn % NBUF
            descs[bn] = pltpu.async_copy(
                table_hbm.at[idx_v.at[pl.ds(kn * CHUNK, CHUNK)]],
                rows[bn],
                sems[bn],
            )
        descs[b].wait()
        pltpu.sync_copy(rows[b], out_hbm.at[pl.ds(base + k * CHUNK, CHUNK)])


@jax.jit
def _sc_gather(table, idx):
    mesh = plsc.VectorSubcoreMesh(core_axis_name="c", subcore_axis_name="s")
    run = pl.kernel(
        _sc_gather_body,
        mesh=mesh,
        out_type=jax.ShapeDtypeStruct((N_IDX, EMB), jnp.float32),
        scratch_types=(
            [pltpu.VMEM((B_PER_W,), jnp.int32)]
            + [pltpu.VMEM((CHUNK, EMB), jnp.float32) for _ in range(NBUF)]
            + [pltpu.SemaphoreType.DMA for _ in range(NBUF)]
        ),
    )
    return run(table, idx)


def _mlp_body(rows_ref, w1_ref, b1_ref, w2_ref, b2_ref, out_ref):
    acc = b1_ref[...] + jnp.dot(
        rows_ref[0], w1_ref[0], preferred_element_type=jnp.float32
    )
    for w in range(1, WINDOW):
        acc = acc + jnp.dot(
            rows_ref[w], w1_ref[w], preferred_element_type=jnp.float32
        )
    h = jnp.tanh(acc)
    out = jnp.dot(h, w2_ref[...], preferred_element_type=jnp.float32) + b2_ref[...]
    out_ref[...] = out[:, :N_TAGS]


@jax.jit
def _mlp(rows3, W13, b1, W2p, b2p):
    return pl.pallas_call(
        _mlp_body,
        grid=(BATCH // BM,),
        in_specs=[
            pl.BlockSpec((WINDOW, BM, EMB), lambda i: (0, i, 0)),
            pl.BlockSpec((WINDOW, EMB, HIDDEN), lambda i: (0, 0, 0)),
            pl.BlockSpec((1, HIDDEN), lambda i: (0, 0)),
            pl.BlockSpec((HIDDEN, 128), lambda i: (0, 0)),
            pl.BlockSpec((1, 128), lambda i: (0, 0)),
        ],
        out_specs=pl.BlockSpec((BM, N_TAGS), lambda i: (i, 0)),
        out_shape=jax.ShapeDtypeStruct((BATCH, N_TAGS), jnp.float32),
    )(rows3, W13, b1, W2p, b2p)


def kernel(x, table, W1, b1, W2, b2):
    idx = x.astype(jnp.int32).T.reshape(-1)
    rows = _sc_gather(table, idx)
    rows3 = rows.reshape(WINDOW, BATCH, EMB)
    W13 = W1.reshape(WINDOW, EMB, HIDDEN)
    W2p = jnp.pad(W2, ((0, 0), (0, 128 - N_TAGS)))
    b2p = jnp.pad(b2, (0, 128 - N_TAGS))
    return _mlp(rows3, W13, b1.reshape(1, -1), W2p, b2p.reshape(1, -1))


# speedup vs baseline: 1.1121x; 1.0574x over previous
import functools

import jax
import jax.numpy as jnp
from jax import lax
from jax.experimental import pallas as pl
from jax.experimental.pallas import tpu as pltpu
from jax.experimental.pallas import tpu_sc as plsc

VOCAB = 1000000
EMB = 128
WINDOW = 5
HIDDEN = 256
N_TAGS = 50
BATCH = 16384

N_IDX = BATCH * WINDOW
NW = 32
B_PER_W = N_IDX // NW
CHUNK = 160
N_CHUNKS = B_PER_W // CHUNK
NBUF = 4
BM = 2048


def _sc_gather_body(table_hbm, idx_hbm, out_hbm, idx_v, *bufs):
    rows = bufs[:NBUF]
    sems = bufs[NBUF:]
    c = lax.axis_index("c")
    s = lax.axis_index("s")
    wid = s * 2 + c
    base = wid * B_PER_W
    pltpu.sync_copy(idx_hbm.at[pl.ds(base, B_PER_W)], idx_v)
    descs = [None] * NBUF
    for k in range(NBUF - 1):
        descs[k] = pltpu.async_copy(
            table_hbm.at[idx_v.at[pl.ds(k * CHUNK, CHUNK)]], rows[k], sems[k]
        )
    for k in range(N_CHUNKS):
        b = k % NBUF
        kn = k + NBUF - 1
        if kn < N_CHUNKS:
            bn = kn % NBUF
            descs[bn] = pltpu.async_copy(
                table_hbm.at[idx_v.at[pl.ds(kn * CHUNK, CHUNK)]],
                rows[bn],
                sems[bn],
            )
        descs[b].wait()
        pltpu.sync_copy(rows[b], out_hbm.at[pl.ds(base + k * CHUNK, CHUNK)])


@jax.jit
def _sc_gather(table, idx):
    mesh = plsc.VectorSubcoreMesh(core_axis_name="c", subcore_axis_name="s")
    run = pl.kernel(
        _sc_gather_body,
        mesh=mesh,
        out_type=jax.ShapeDtypeStruct((N_IDX, EMB), jnp.float32),
        scratch_types=(
            [pltpu.VMEM((B_PER_W,), jnp.int32)]
            + [pltpu.VMEM((CHUNK, EMB), jnp.float32) for _ in range(NBUF)]
            + [pltpu.SemaphoreType.DMA for _ in range(NBUF)]
        ),
    )
    return run(table, idx)


def _mlp_body(rows_ref, w1_ref, b1_ref, w2_ref, b2_ref, out_ref):
    acc = b1_ref[...] + jnp.dot(
        rows_ref[0], w1_ref[0], preferred_element_type=jnp.float32
    )
    for w in range(1, WINDOW):
        acc = acc + jnp.dot(
            rows_ref[w], w1_ref[w], preferred_element_type=jnp.float32
        )
    h = jnp.tanh(acc)
    out = jnp.dot(h, w2_ref[...], preferred_element_type=jnp.float32) + b2_ref[...]
    out_ref[...] = out[:, :N_TAGS]


@jax.jit
def _mlp(rows3, W13, b1, W2p, b2p):
    return pl.pallas_call(
        _mlp_body,
        grid=(BATCH // BM,),
        in_specs=[
            pl.BlockSpec((WINDOW, BM, EMB), lambda i: (0, i, 0)),
            pl.BlockSpec((WINDOW, EMB, HIDDEN), lambda i: (0, 0, 0)),
            pl.BlockSpec((1, HIDDEN), lambda i: (0, 0)),
            pl.BlockSpec((HIDDEN, 128), lambda i: (0, 0)),
            pl.BlockSpec((1, 128), lambda i: (0, 0)),
        ],
        out_specs=pl.BlockSpec((BM, N_TAGS), lambda i: (i, 0)),
        out_shape=jax.ShapeDtypeStruct((BATCH, N_TAGS), jnp.float32),
    )(rows3, W13, b1, W2p, b2p)


def kernel(x, table, W1, b1, W2, b2):
    idx = x.astype(jnp.int32).T.reshape(-1)
    rows = _sc_gather(table, idx)
    rows3 = rows.reshape(WINDOW, BATCH, EMB)
    W13 = W1.reshape(WINDOW, EMB, HIDDEN)
    W2p = jnp.pad(W2, ((0, 0), (0, 128 - N_TAGS)))
    b2p = jnp.pad(b2, (0, 128 - N_TAGS))
    return _mlp(rows3, W13, b1.reshape(1, -1), W2p, b2p.reshape(1, -1))


# speedup vs baseline: 1.1290x; 1.0152x over previous
import functools

import jax
import jax.numpy as jnp
from jax import lax
from jax.experimental import pallas as pl
from jax.experimental.pallas import tpu as pltpu
from jax.experimental.pallas import tpu_sc as plsc

VOCAB = 1000000
EMB = 128
WINDOW = 5
HIDDEN = 256
N_TAGS = 50
BATCH = 16384

N_IDX = BATCH * WINDOW
NW = 32
B_PER_W = N_IDX // NW
CHUNK = 160
N_CHUNKS = B_PER_W // CHUNK
NBUF = 4
BM = 4096


def _sc_gather_body(table_hbm, idx_hbm, out_hbm, idx_v, *bufs):
    rows = bufs[:NBUF]
    sems = bufs[NBUF:]
    c = lax.axis_index("c")
    s = lax.axis_index("s")
    wid = s * 2 + c
    base = wid * B_PER_W
    pltpu.sync_copy(idx_hbm.at[pl.ds(base, B_PER_W)], idx_v)
    descs = [None] * NBUF
    for k in range(NBUF - 1):
        descs[k] = pltpu.async_copy(
            table_hbm.at[idx_v.at[pl.ds(k * CHUNK, CHUNK)]], rows[k], sems[k]
        )
    for k in range(N_CHUNKS):
        b = k % NBUF
        kn = k + NBUF - 1
        if kn < N_CHUNKS:
            bn = kn % NBUF
            descs[bn] = pltpu.async_copy(
                table_hbm.at[idx_v.at[pl.ds(kn * CHUNK, CHUNK)]],
                rows[bn],
                sems[bn],
            )
        descs[b].wait()
        pltpu.sync_copy(rows[b], out_hbm.at[pl.ds(base + k * CHUNK, CHUNK)])


@jax.jit
def _sc_gather(table, idx):
    mesh = plsc.VectorSubcoreMesh(core_axis_name="c", subcore_axis_name="s")
    run = pl.kernel(
        _sc_gather_body,
        mesh=mesh,
        out_type=jax.ShapeDtypeStruct((N_IDX, EMB), jnp.float32),
        scratch_types=(
            [pltpu.VMEM((B_PER_W,), jnp.int32)]
            + [pltpu.VMEM((CHUNK, EMB), jnp.float32) for _ in range(NBUF)]
            + [pltpu.SemaphoreType.DMA for _ in range(NBUF)]
        ),
    )
    return run(table, idx)


def _mlp_body(rows_ref, w1_ref, b1_ref, w2_ref, b2_ref, out_ref):
    acc = b1_ref[...] + jnp.dot(
        rows_ref[0], w1_ref[0], preferred_element_type=jnp.float32
    )
    for w in range(1, WINDOW):
        acc = acc + jnp.dot(
            rows_ref[w], w1_ref[w], preferred_element_type=jnp.float32
        )
    h = jnp.tanh(acc)
    out = jnp.dot(h, w2_ref[...], preferred_element_type=jnp.float32) + b2_ref[...]
    out_ref[...] = out[:, :N_TAGS]


@jax.jit
def _mlp(rows3, W13, b1, W2p, b2p):
    return pl.pallas_call(
        _mlp_body,
        grid=(BATCH // BM,),
        in_specs=[
            pl.BlockSpec((WINDOW, BM, EMB), lambda i: (0, i, 0)),
            pl.BlockSpec((WINDOW, EMB, HIDDEN), lambda i: (0, 0, 0)),
            pl.BlockSpec((1, HIDDEN), lambda i: (0, 0)),
            pl.BlockSpec((HIDDEN, 128), lambda i: (0, 0)),
            pl.BlockSpec((1, 128), lambda i: (0, 0)),
        ],
        out_specs=pl.BlockSpec((BM, N_TAGS), lambda i: (i, 0)),
        out_shape=jax.ShapeDtypeStruct((BATCH, N_TAGS), jnp.float32),
    )(rows3, W13, b1, W2p, b2p)


def kernel(x, table, W1, b1, W2, b2):
    idx = x.astype(jnp.int32).T.reshape(-1)
    rows = _sc_gather(table, idx)
    rows3 = rows.reshape(WINDOW, BATCH, EMB)
    W13 = W1.reshape(WINDOW, EMB, HIDDEN)
    W2p = jnp.pad(W2, ((0, 0), (0, 128 - N_TAGS)))
    b2p = jnp.pad(b2, (0, 128 - N_TAGS))
    return _mlp(rows3, W13, b1.reshape(1, -1), W2p, b2p.reshape(1, -1))
